# baseline (speedup 1.0000x reference)
import functools

import jax
import jax.numpy as jnp
from jax import lax
from jax.experimental import pallas as pl
from jax.experimental.pallas import tpu as pltpu
from jax.experimental.pallas import tpu_sc as plsc

NUM_GRAPHS = 64
E_ROWS = 320000
N_ROWS = 10000
E_FEATS = 16
X_FEATS = 128
OUT_FEATS = 128

NC = 2
NS = 16
NW = NC * NS

N_CH = 80
N_NCH = N_ROWS // N_CH
N_ITERS = (N_NCH + NW - 1) // NW

E_GRID = 50
E_BLK = E_ROWS // E_GRID


def _sc_body(x_hbm, nids_hbm, px_hbm, nbuf, nids, zbuf, nacc_sp):
    cid = lax.axis_index("c")
    sid = lax.axis_index("s")
    wid = sid * NC + cid

    zero16 = jnp.zeros((16,), jnp.float32)

    def _zrow(g, _):
        for k in range(X_FEATS // 16):
            zbuf[g, pl.ds(k * 16, 16)] = zero16
        return 0

    lax.fori_loop(0, NUM_GRAPHS, _zrow, 0)
    pltpu.sync_copy(zbuf, nacc_sp.at[sid])

    def _nchunk(it, _):
        c = wid + it * NW

        @pl.when(c < N_NCH)
        def _do():
            pltpu.sync_copy(x_hbm.at[pl.ds(c * N_CH, N_CH)], nbuf)
            pltpu.sync_copy(nids_hbm.at[pl.ds(c, 1)], nids)
            pltpu.sync_copy(nbuf, nacc_sp.at[sid].at[nids.at[0]], add=True)

        return 0

    lax.fori_loop(0, N_ITERS, _nchunk, 0)

    pltpu.sync_copy(nacc_sp.at[sid], px_hbm.at[wid])


_sc_nodes = pl.kernel(
    _sc_body,
    out_type=jax.ShapeDtypeStruct((NW, NUM_GRAPHS, X_FEATS), jnp.float32),
    mesh=plsc.VectorSubcoreMesh(core_axis_name="c", subcore_axis_name="s"),
    scratch_types=[
        pltpu.VMEM((N_CH, X_FEATS), jnp.float32),
        pltpu.VMEM((1, N_CH), jnp.int32),
        pltpu.VMEM((NUM_GRAPHS, X_FEATS), jnp.float32),
        pltpu.VMEM_SHARED((NS, NUM_GRAPHS, X_FEATS), jnp.float32),
    ],
    compiler_params=pltpu.CompilerParams(use_tc_tiling_on_sc=False),
)


def _edges_body(eids_ref, et_ref, acc_ref, acc):
    step = pl.program_id(0)

    @pl.when(step == 0)
    def _init():
        acc[...] = jnp.zeros_like(acc)

    gids = lax.broadcasted_iota(jnp.int32, (E_BLK, NUM_GRAPHS), 1)
    onehot = (eids_ref[0, 0, :][:, None] == gids).astype(jnp.bfloat16)
    acc[...] += jax.lax.dot(et_ref[...].astype(jnp.bfloat16), onehot,
                            preferred_element_type=jnp.float32)

    @pl.when(step == E_GRID - 1)
    def _done():
        acc_ref[...] = acc[...]


def _tc_edges(et, eids3):
    return pl.pallas_call(
        _edges_body,
        grid=(E_GRID,),
        in_specs=[
            pl.BlockSpec((1, 1, E_BLK), lambda i: (i, 0, 0)),
            pl.BlockSpec((E_FEATS, E_BLK), lambda i: (0, i)),
        ],
        out_specs=pl.BlockSpec((E_FEATS, NUM_GRAPHS), lambda i: (0, 0)),
        out_shape=jax.ShapeDtypeStruct((E_FEATS, NUM_GRAPHS), jnp.float32),
        scratch_shapes=[pltpu.VMEM((E_FEATS, NUM_GRAPHS), jnp.float32)],
        compiler_params=pltpu.CompilerParams(
            dimension_semantics=("arbitrary",),
        ),
    )(eids3, et)


def _lin_body(aet_ref, px_ref, g_ref, w_ref, b_ref, out_ref):
    agg_x = jnp.sum(px_ref[...], axis=0)
    w = w_ref[...]
    out = jax.lax.dot_general(aet_ref[...], w[0:E_FEATS, :],
                              (((0,), (0,)), ((), ())),
                              preferred_element_type=jnp.float32)
    out += jax.lax.dot(agg_x, w[E_FEATS:E_FEATS + X_FEATS, :],
                       preferred_element_type=jnp.float32)
    out += jax.lax.dot(g_ref[...], w[E_FEATS + X_FEATS:, :],
                       preferred_element_type=jnp.float32)
    out_ref[...] = out + b_ref[0, :][None, :]


def _linear(agg_et, px, global_attr, W, b2):
    return pl.pallas_call(
        _lin_body,
        out_shape=jax.ShapeDtypeStruct((NUM_GRAPHS, OUT_FEATS), jnp.float32),
    )(agg_et, px, global_attr, W, b2)


@jax.jit
def _global_block(x, e, global_attr, node_ids, edge_ids, W, b):
    et = e.T
    eids3 = edge_ids.astype(jnp.int32).reshape(E_GRID, 1, E_BLK)
    nids2 = node_ids.astype(jnp.int32).reshape(N_NCH, N_CH)
    px = _sc_nodes(x, nids2)
    agg_et = _tc_edges(et, eids3)
    return _linear(agg_et, px, global_attr, W, b.reshape(1, OUT_FEATS))


def kernel(x, e, global_attr, node_graph_ids, edge_graph_ids, W, b):
    return _global_block(x, e, global_attr, node_graph_ids, edge_graph_ids,
                         W, b)


# speedup vs baseline: 1.2989x; 1.2989x over previous
import functools

import jax
import jax.numpy as jnp
from jax import lax
from jax.experimental import pallas as pl
from jax.experimental.pallas import tpu as pltpu
from jax.experimental.pallas import tpu_sc as plsc

NUM_GRAPHS = 64
E_ROWS = 320000
N_ROWS = 10000
E_FEATS = 16
X_FEATS = 128
OUT_FEATS = 128

NC = 2
NS = 16
NW = NC * NS

N_CH = 80
N_NCH = N_ROWS // N_CH
N_ITERS = (N_NCH + NW - 1) // NW

E_GRID = 50
E_BLK = E_ROWS // E_GRID


def _sc_body(x_hbm, nids_hbm, px_hbm, nbuf, nids, zbuf, nacc_sp):
    cid = lax.axis_index("c")
    sid = lax.axis_index("s")
    wid = sid * NC + cid

    zero16 = jnp.zeros((16,), jnp.float32)

    def _zrow(g, _):
        for k in range(X_FEATS // 16):
            zbuf[g, pl.ds(k * 16, 16)] = zero16
        return 0

    lax.fori_loop(0, NUM_GRAPHS, _zrow, 0)
    pltpu.sync_copy(zbuf, nacc_sp.at[sid])

    def _nchunk(it, _):
        c = wid + it * NW

        @pl.when(c < N_NCH)
        def _do():
            pltpu.sync_copy(x_hbm.at[pl.ds(c * N_CH, N_CH)], nbuf)
            pltpu.sync_copy(nids_hbm.at[pl.ds(c, 1)], nids)
            pltpu.sync_copy(nbuf, nacc_sp.at[sid].at[nids.at[0]], add=True)

        return 0

    lax.fori_loop(0, N_ITERS, _nchunk, 0)

    pltpu.sync_copy(nacc_sp.at[sid], px_hbm.at[wid])


_sc_nodes = pl.kernel(
    _sc_body,
    out_type=jax.ShapeDtypeStruct((NW, NUM_GRAPHS, X_FEATS), jnp.float32),
    mesh=plsc.VectorSubcoreMesh(core_axis_name="c", subcore_axis_name="s"),
    scratch_types=[
        pltpu.VMEM((N_CH, X_FEATS), jnp.float32),
        pltpu.VMEM((1, N_CH), jnp.int32),
        pltpu.VMEM((NUM_GRAPHS, X_FEATS), jnp.float32),
        pltpu.VMEM_SHARED((NS, NUM_GRAPHS, X_FEATS), jnp.float32),
    ],
    compiler_params=pltpu.CompilerParams(use_tc_tiling_on_sc=False),
)


def _edges_body(eids_ref, et_ref, acc_ref, acc):
    step = pl.program_id(0)

    @pl.when(step == 0)
    def _init():
        acc[...] = jnp.zeros_like(acc)

    gids = lax.broadcasted_iota(jnp.int32, (NUM_GRAPHS, E_BLK), 0)
    onehot = (eids_ref[0, 0, :][None, :] == gids).astype(jnp.bfloat16)
    etr = jnp.transpose(et_ref[...].astype(jnp.bfloat16), (1, 0))
    acc[...] += jax.lax.dot(onehot, etr,
                            preferred_element_type=jnp.float32)

    @pl.when(step == E_GRID - 1)
    def _done():
        acc_ref[...] = acc[...]


def _tc_edges(et, eids3):
    return pl.pallas_call(
        _edges_body,
        grid=(E_GRID,),
        in_specs=[
            pl.BlockSpec((1, 1, E_BLK), lambda i: (i, 0, 0)),
            pl.BlockSpec((E_FEATS, E_BLK), lambda i: (0, i)),
        ],
        out_specs=pl.BlockSpec((NUM_GRAPHS, E_FEATS), lambda i: (0, 0)),
        out_shape=jax.ShapeDtypeStruct((NUM_GRAPHS, E_FEATS), jnp.float32),
        scratch_shapes=[pltpu.VMEM((NUM_GRAPHS, E_FEATS), jnp.float32)],
        compiler_params=pltpu.CompilerParams(
            dimension_semantics=("arbitrary",),
        ),
    )(eids3, et)


def _lin_body(aet_ref, px_ref, g_ref, w_ref, b_ref, out_ref):
    agg_x = jnp.sum(px_ref[...], axis=0)
    w = w_ref[...]
    out = jax.lax.dot(aet_ref[...], w[0:E_FEATS, :],
                      preferred_element_type=jnp.float32)
    out += jax.lax.dot(agg_x, w[E_FEATS:E_FEATS + X_FEATS, :],
                       preferred_element_type=jnp.float32)
    out += jax.lax.dot(g_ref[...], w[E_FEATS + X_FEATS:, :],
                       preferred_element_type=jnp.float32)
    out_ref[...] = out + b_ref[0, :][None, :]


def _linear(agg_et, px, global_attr, W, b2):
    return pl.pallas_call(
        _lin_body,
        out_shape=jax.ShapeDtypeStruct((NUM_GRAPHS, OUT_FEATS), jnp.float32),
    )(agg_et, px, global_attr, W, b2)


@jax.jit
def _global_block(x, e, global_attr, node_ids, edge_ids, W, b):
    et = e.T
    eids3 = edge_ids.astype(jnp.int32).reshape(E_GRID, 1, E_BLK)
    nids2 = node_ids.astype(jnp.int32).reshape(N_NCH, N_CH)
    px = _sc_nodes(x, nids2)
    agg_et = _tc_edges(et, eids3)
    return _linear(agg_et, px, global_attr, W, b.reshape(1, OUT_FEATS))


def kernel(x, e, global_attr, node_graph_ids, edge_graph_ids, W, b):
    return _global_block(x, e, global_attr, node_graph_ids, edge_graph_ids,
                         W, b)


# speedup vs baseline: 1.6498x; 1.2701x over previous
import functools

import jax
import jax.numpy as jnp
from jax import lax
from jax.experimental import pallas as pl
from jax.experimental.pallas import tpu as pltpu
from jax.experimental.pallas import tpu_sc as plsc

NUM_GRAPHS = 64
E_ROWS = 320000
N_ROWS = 10000
E_FEATS = 16
X_FEATS = 128
OUT_FEATS = 128

NC = 2
NS = 16
NW = NC * NS

N_CH = 80
N_NCH = N_ROWS // N_CH
N_ITERS = (N_NCH + NW - 1) // NW

E_GRID = 25
E_BLK = E_ROWS // E_GRID


def _sc_body(x_hbm, nids_hbm, px_hbm, nbuf, nids, zbuf, nacc_sp):
    cid = lax.axis_index("c")
    sid = lax.axis_index("s")
    wid = sid * NC + cid

    zero16 = jnp.zeros((16,), jnp.float32)

    def _zrow(g, _):
        for k in range(X_FEATS // 16):
            zbuf[g, pl.ds(k * 16, 16)] = zero16
        return 0

    lax.fori_loop(0, NUM_GRAPHS, _zrow, 0)
    pltpu.sync_copy(zbuf, nacc_sp.at[sid])

    def _nchunk(it, _):
        c = wid + it * NW

        @pl.when(c < N_NCH)
        def _do():
            pltpu.sync_copy(x_hbm.at[pl.ds(c * N_CH, N_CH)], nbuf)
            pltpu.sync_copy(nids_hbm.at[pl.ds(c, 1)], nids)
            pltpu.sync_copy(nbuf, nacc_sp.at[sid].at[nids.at[0]], add=True)

        return 0

    lax.fori_loop(0, N_ITERS, _nchunk, 0)

    pltpu.sync_copy(nacc_sp.at[sid], px_hbm.at[wid])


_sc_nodes = pl.kernel(
    _sc_body,
    out_type=jax.ShapeDtypeStruct((NW, NUM_GRAPHS, X_FEATS), jnp.float32),
    mesh=plsc.VectorSubcoreMesh(core_axis_name="c", subcore_axis_name="s"),
    scratch_types=[
        pltpu.VMEM((N_CH, X_FEATS), jnp.float32),
        pltpu.VMEM((1, N_CH), jnp.int32),
        pltpu.VMEM((NUM_GRAPHS, X_FEATS), jnp.float32),
        pltpu.VMEM_SHARED((NS, NUM_GRAPHS, X_FEATS), jnp.float32),
    ],
    compiler_params=pltpu.CompilerParams(use_tc_tiling_on_sc=False),
)


def _edges_body(eids_ref, et_ref, acc_ref, acc):
    step = pl.program_id(0)

    @pl.when(step == 0)
    def _init():
        acc[...] = jnp.zeros_like(acc)

    gids = lax.broadcasted_iota(jnp.int32, (NUM_GRAPHS, E_BLK), 0)
    onehot = (eids_ref[0, 0, :][None, :] == gids).astype(jnp.bfloat16)
    etr = jnp.transpose(et_ref[...].astype(jnp.bfloat16), (1, 0))
    acc[...] += jax.lax.dot(onehot, etr,
                            preferred_element_type=jnp.float32)

    @pl.when(step == E_GRID - 1)
    def _done():
        acc_ref[...] = acc[...]


def _tc_edges(et, eids3):
    return pl.pallas_call(
        _edges_body,
        grid=(E_GRID,),
        in_specs=[
            pl.BlockSpec((1, 1, E_BLK), lambda i: (i, 0, 0)),
            pl.BlockSpec((E_FEATS, E_BLK), lambda i: (0, i)),
        ],
        out_specs=pl.BlockSpec((NUM_GRAPHS, E_FEATS), lambda i: (0, 0)),
        out_shape=jax.ShapeDtypeStruct((NUM_GRAPHS, E_FEATS), jnp.float32),
        scratch_shapes=[pltpu.VMEM((NUM_GRAPHS, E_FEATS), jnp.float32)],
        compiler_params=pltpu.CompilerParams(
            dimension_semantics=("arbitrary",),
        ),
    )(eids3, et)


def _lin_body(aet_ref, px_ref, g_ref, w_ref, b_ref, out_ref):
    agg_x = jnp.sum(px_ref[...], axis=0)
    w = w_ref[...]
    out = jax.lax.dot(aet_ref[...], w[0:E_FEATS, :],
                      preferred_element_type=jnp.float32)
    out += jax.lax.dot(agg_x, w[E_FEATS:E_FEATS + X_FEATS, :],
                       preferred_element_type=jnp.float32)
    out += jax.lax.dot(g_ref[...], w[E_FEATS + X_FEATS:, :],
                       preferred_element_type=jnp.float32)
    out_ref[...] = out + b_ref[0, :][None, :]


def _linear(agg_et, px, global_attr, W, b2):
    return pl.pallas_call(
        _lin_body,
        out_shape=jax.ShapeDtypeStruct((NUM_GRAPHS, OUT_FEATS), jnp.float32),
    )(agg_et, px, global_attr, W, b2)


@jax.jit
def _global_block(x, e, global_attr, node_ids, edge_ids, W, b):
    et = e.T
    eids3 = edge_ids.astype(jnp.int32).reshape(E_GRID, 1, E_BLK)
    nids2 = node_ids.astype(jnp.int32).reshape(N_NCH, N_CH)
    px = _sc_nodes(x, nids2)
    agg_et = _tc_edges(et, eids3)
    return _linear(agg_et, px, global_attr, W, b.reshape(1, OUT_FEATS))


def kernel(x, e, global_attr, node_graph_ids, edge_graph_ids, W, b):
    return _global_block(x, e, global_attr, node_graph_ids, edge_graph_ids,
                         W, b)
